# Initial kernel scaffold; baseline (speedup 1.0000x reference)
import jax
import jax.numpy as jnp
from jax.experimental import pallas as pl


def kernel(x, edge_index, Wa_w, Wa_b, Wm_w, Wm_b, Wr_w, Wr_b, bn_gamma, bn_beta, bn_mean, bn_var):
    raise NotImplementedError("write your pallas kernel here")


# speedup vs baseline: 8.8026x; 8.8026x over previous
import functools

import jax
import jax.numpy as jnp
from jax import lax
from jax.experimental import pallas as pl
from jax.experimental.pallas import tpu as pltpu
from jax.experimental.pallas import tpu_sc as plsc

_N = 10000
_E = 320000
_D = 128
_EPS = 1e-5
_NC, _NS, _L = 2, 16, 16
_NW = _NC * _NS
_EPW = _E // _NW
_CH = 80
_NCHUNK = _EPW // _CH
_CPB = 25
_NBLK = _NCHUNK // _CPB
_RPT = 632
_NPAD = _NS * _RPT
_BN = 1000
_GRID = _N // _BN


def _tc_pre_body(x_ref, wmt_ref, wmb_ref, wat_ref, wab_ref, wrt_ref, wrb_ref,
                 gam_ref, bet_ref, mu_ref, var_ref,
                 xm_ref, asd_ref, base_ref):
    x = x_ref[...]
    k = gam_ref[...] * lax.rsqrt(var_ref[...] + _EPS)
    xm = jnp.dot(x, wmt_ref[...], preferred_element_type=jnp.float32)
    xm_ref[...] = (xm + wmb_ref[...]) * k
    asd_ref[...] = jnp.dot(x, wat_ref[...], preferred_element_type=jnp.float32) + wab_ref[...]
    xr = jnp.dot(x, wrt_ref[...], preferred_element_type=jnp.float32)
    base_ref[...] = (xr + wrb_ref[...] - mu_ref[...]) * k + bet_ref[...]


def _tc_pre(x, wmt, wmb, wat, wab, wrt, wrb, gam, bet, mu, var):
    full = lambda shape: pl.BlockSpec(shape, lambda i: (0, 0))
    blk = lambda shape: pl.BlockSpec(shape, lambda i: (i, 0))
    return pl.pallas_call(
        _tc_pre_body,
        grid=(_GRID,),
        in_specs=[
            blk((_BN, _D)),
            full((_D, _D)), full((1, _D)),
            full((_D, 2)), full((1, 2)),
            full((_D, _D)), full((1, _D)),
            full((1, _D)), full((1, _D)), full((1, _D)), full((1, _D)),
        ],
        out_specs=[blk((_BN, _D)), blk((_BN, 2)), blk((_BN, _D))],
        out_shape=[
            jax.ShapeDtypeStruct((_N, _D), jnp.float32),
            jax.ShapeDtypeStruct((_N, 2), jnp.float32),
            jax.ShapeDtypeStruct((_N, _D), jnp.float32),
        ],
    )(x, wmt, wmb, wat, wab, wrt, wrb, gam, bet, mu, var)


_GDN = lax.GatherDimensionNumbers(offset_dims=(), collapsed_slice_dims=(0,),
                                  start_index_map=(0,))


def _lane_bcast(vec, t):
    idx = jnp.full((_L, 1), t, jnp.int32)
    return lax.gather(vec, idx, _GDN, slice_sizes=(1,),
                      mode=lax.GatherScatterMode.PROMISE_IN_BOUNDS)


def _sc_edges(xm, asd_flat, eidx):
    mesh = plsc.VectorSubcoreMesh(core_axis_name="c", subcore_axis_name="s",
                                  num_cores=_NC, num_subcores=_NS)

    @functools.partial(
        pl.kernel,
        out_type=jax.ShapeDtypeStruct((_NC, _NPAD, _D), jnp.float32),
        mesh=mesh,
        scratch_types=[
            pltpu.VMEM((2 * _N,), jnp.float32),
            pltpu.VMEM((_CPB, _CH), jnp.int32),
            pltpu.VMEM((_CPB, _CH), jnp.int32),
            pltpu.VMEM((_CH, _D), jnp.float32),
            pltpu.VMEM_SHARED((_NPAD, _D), jnp.float32),
            pltpu.SemaphoreType.DMA,
        ],
        compiler_params=pltpu.CompilerParams(needs_layout_passes=False),
    )
    def k(xm_hbm, asd_hbm, eidx_hbm, out_hbm,
          asd_v, src_v, dst_v, rows_v, agg_sh, sem):
        c = lax.axis_index("c")
        s = lax.axis_index("s")
        wid = s * _NC + c

        pltpu.sync_copy(asd_hbm, asd_v)

        zero = jnp.zeros((_L,), jnp.float32)
        for r in range(_CH):
            for cc in range(_D // _L):
                rows_v[r, pl.ds(cc * _L, _L)] = zero
        zbase = s * _RPT
        for i in range(7):
            pltpu.sync_copy(rows_v, agg_sh.at[pl.ds(zbase + i * _CH, _CH)])
        rem = _RPT - 7 * _CH
        pltpu.sync_copy(rows_v.at[pl.ds(0, rem)],
                        agg_sh.at[pl.ds(zbase + 7 * _CH, rem)])
        plsc.subcore_barrier()

        def block(jj, carry0):
            pltpu.sync_copy(eidx_hbm.at[0, wid, jj], src_v)
            pltpu.sync_copy(eidx_hbm.at[1, wid, jj], dst_v)

            def chunk(j, carry):
                pltpu.async_copy(xm_hbm.at[src_v.at[j]], rows_v, sem).wait()
                jf = jnp.full((_L,), j, jnp.int32)
                iota = lax.iota(jnp.int32, _L)
                for v in range(_CH // _L):
                    cols = iota + (v * _L)
                    si = plsc.load_gather(src_v, [jf, cols])
                    di = plsc.load_gather(dst_v, [jf, cols])
                    ga = plsc.load_gather(asd_v, [si * 2])
                    gb = plsc.load_gather(asd_v, [di * 2 + 1])
                    g = 1.0 / (1.0 + jnp.exp(-(ga + gb)))
                    for t in range(_L):
                        gvec = _lane_bcast(g, t)
                        r = v * _L + t
                        for cc in range(_D // _L):
                            csl = pl.ds(cc * _L, _L)
                            rows_v[r, csl] = rows_v[r, csl] * gvec
                pltpu.sync_copy(rows_v, agg_sh.at[dst_v.at[j]], add=True)
                return carry

            lax.fori_loop(0, _CPB, chunk, 0)
            return carry0

        lax.fori_loop(0, _NBLK, block, 0)
        plsc.subcore_barrier()
        pltpu.sync_copy(agg_sh.at[pl.ds(s * _RPT, _RPT)],
                        out_hbm.at[c, pl.ds(s * _RPT, _RPT)])

    return k(xm, asd_flat, eidx)


def _tc_post_body(p_ref, base_ref, out_ref):
    out_ref[...] = jnp.maximum(p_ref[0] + p_ref[1] + base_ref[...], 0.0)


def _tc_post(partials, base):
    return pl.pallas_call(
        _tc_post_body,
        grid=(_GRID,),
        in_specs=[
            pl.BlockSpec((_NC, _BN, _D), lambda i: (0, i, 0)),
            pl.BlockSpec((_BN, _D), lambda i: (i, 0)),
        ],
        out_specs=pl.BlockSpec((_BN, _D), lambda i: (i, 0)),
        out_shape=jax.ShapeDtypeStruct((_N, _D), jnp.float32),
    )(partials, base)


def kernel(x, edge_index, Wa_w, Wa_b, Wm_w, Wm_b, Wr_w, Wr_b,
           bn_gamma, bn_beta, bn_mean, bn_var):
    wmt = Wm_w.T
    wrt = Wr_w.T
    wat = Wa_w.reshape(2, _D).T
    wab = jnp.concatenate([Wa_b, jnp.zeros((1,), jnp.float32)]).reshape(1, 2)
    xm, asd, base = _tc_pre(
        x, wmt, Wm_b.reshape(1, _D), wat, wab, wrt, Wr_b.reshape(1, _D),
        bn_gamma.reshape(1, _D), bn_beta.reshape(1, _D),
        bn_mean.reshape(1, _D), bn_var.reshape(1, _D))
    partials = _sc_edges(xm, asd.reshape(-1),
                         edge_index.reshape(2, _NW, _NBLK, _CPB, _CH))
    return _tc_post(partials, base)


# speedup vs baseline: 11.2293x; 1.2757x over previous
import functools

import jax
import jax.numpy as jnp
from jax import lax
from jax.experimental import pallas as pl
from jax.experimental.pallas import tpu as pltpu
from jax.experimental.pallas import tpu_sc as plsc

_N = 10000
_E = 320000
_D = 128
_EPS = 1e-5
_NC, _NS, _L = 2, 16, 16
_NW = _NC * _NS
_EPW = _E // _NW
_CH = 80
_NCHUNK = _EPW // _CH
_CPB = 25
_NBLK = _NCHUNK // _CPB
_NPAIR = (_CPB - 1) // 2
_RPT = 632
_NPAD = _NS * _RPT
_BN = 1000
_GRID = _N // _BN


def _tc_pre_body(x_ref, wmt_ref, wmb_ref, wat_ref, wab_ref, wrt_ref, wrb_ref,
                 gam_ref, bet_ref, mu_ref, var_ref,
                 xm_ref, asd_ref, base_ref):
    x = x_ref[...]
    k = gam_ref[...] * lax.rsqrt(var_ref[...] + _EPS)
    xm = jnp.dot(x, wmt_ref[...], preferred_element_type=jnp.float32)
    xm_ref[...] = (xm + wmb_ref[...]) * k
    asd_ref[...] = jnp.dot(x, wat_ref[...], preferred_element_type=jnp.float32) + wab_ref[...]
    xr = jnp.dot(x, wrt_ref[...], preferred_element_type=jnp.float32)
    base_ref[...] = (xr + wrb_ref[...] - mu_ref[...]) * k + bet_ref[...]


def _tc_pre(x, wmt, wmb, wat, wab, wrt, wrb, gam, bet, mu, var):
    full = lambda shape: pl.BlockSpec(shape, lambda i: (0, 0))
    blk = lambda shape: pl.BlockSpec(shape, lambda i: (i, 0))
    return pl.pallas_call(
        _tc_pre_body,
        grid=(_GRID,),
        in_specs=[
            blk((_BN, _D)),
            full((_D, _D)), full((1, _D)),
            full((_D, 2)), full((1, 2)),
            full((_D, _D)), full((1, _D)),
            full((1, _D)), full((1, _D)), full((1, _D)), full((1, _D)),
        ],
        out_specs=[blk((_BN, _D)), blk((_BN, 2)), blk((_BN, _D))],
        out_shape=[
            jax.ShapeDtypeStruct((_N, _D), jnp.float32),
            jax.ShapeDtypeStruct((_N, 2), jnp.float32),
            jax.ShapeDtypeStruct((_N, _D), jnp.float32),
        ],
    )(x, wmt, wmb, wat, wab, wrt, wrb, gam, bet, mu, var)


_GDN = lax.GatherDimensionNumbers(offset_dims=(), collapsed_slice_dims=(0,),
                                  start_index_map=(0,))


def _lane_bcast(vec, t):
    idx = jnp.full((_L, 1), t, jnp.int32)
    return lax.gather(vec, idx, _GDN, slice_sizes=(1,),
                      mode=lax.GatherScatterMode.PROMISE_IN_BOUNDS)


def _sc_edges(xm, a_src, a_dst, eidx):
    mesh = plsc.VectorSubcoreMesh(core_axis_name="c", subcore_axis_name="s",
                                  num_cores=_NC, num_subcores=_NS)

    @functools.partial(
        pl.kernel,
        out_type=jax.ShapeDtypeStruct((_NC, _NPAD, _D), jnp.float32),
        mesh=mesh,
        scratch_types=[
            pltpu.VMEM((_CPB, _CH), jnp.int32),
            pltpu.VMEM((_CPB, _CH), jnp.int32),
            pltpu.VMEM((2, _CH, _D), jnp.float32),
            pltpu.VMEM((2, _CH), jnp.float32),
            pltpu.VMEM((2, _CH), jnp.float32),
            pltpu.VMEM_SHARED((_NPAD, _D), jnp.float32),
            pltpu.SemaphoreType.DMA,
            pltpu.SemaphoreType.DMA,
            pltpu.SemaphoreType.DMA,
            pltpu.SemaphoreType.DMA,
        ],
        compiler_params=pltpu.CompilerParams(needs_layout_passes=False),
    )
    def k(xm_hbm, as_hbm, ad_hbm, eidx_hbm, out_hbm,
          src_v, dst_v, rows_v, ga_v, gb_v, agg_sh,
          gsem0, gsem1, ssem0, ssem1):
        c = lax.axis_index("c")
        s = lax.axis_index("s")
        wid = s * _NC + c
        gsems = (gsem0, gsem1)
        ssems = (ssem0, ssem1)

        def issue_gathers(j, p):
            pltpu.async_copy(xm_hbm.at[src_v.at[j]], rows_v.at[p], gsems[p])
            pltpu.async_copy(as_hbm.at[src_v.at[j]], ga_v.at[p], gsems[p])
            pltpu.async_copy(ad_hbm.at[dst_v.at[j]], gb_v.at[p], gsems[p])

        def wait_gathers(j, p):
            pltpu.make_async_copy(xm_hbm.at[src_v.at[j]], rows_v.at[p],
                                  gsems[p]).wait()
            pltpu.make_async_copy(as_hbm.at[src_v.at[j]], ga_v.at[p],
                                  gsems[p]).wait()
            pltpu.make_async_copy(ad_hbm.at[dst_v.at[j]], gb_v.at[p],
                                  gsems[p]).wait()

        def issue_scatter(j, p):
            pltpu.async_copy(rows_v.at[p], agg_sh.at[dst_v.at[j]], ssems[p],
                             add=True)

        def wait_scatter(j, p):
            pltpu.make_async_copy(rows_v.at[p], agg_sh.at[dst_v.at[j]],
                                  ssems[p]).wait()

        def compute(p):
            for v in range(_CH // _L):
                sl = pl.ds(v * _L, _L)
                g = 1.0 / (1.0 + jnp.exp(-(ga_v[p, sl] + gb_v[p, sl])))
                for t in range(_L):
                    gvec = _lane_bcast(g, t)
                    r = v * _L + t
                    for cc in range(_D // _L):
                        csl = pl.ds(cc * _L, _L)
                        rows_v[p, r, csl] = rows_v[p, r, csl] * gvec

        zero = jnp.zeros((_L,), jnp.float32)
        for r in range(_CH):
            for cc in range(_D // _L):
                rows_v[0, r, pl.ds(cc * _L, _L)] = zero
        zbase = s * _RPT
        for i in range(7):
            pltpu.sync_copy(rows_v.at[0], agg_sh.at[pl.ds(zbase + i * _CH, _CH)])
        rem = _RPT - 7 * _CH
        pltpu.sync_copy(rows_v.at[0].at[pl.ds(0, rem)],
                        agg_sh.at[pl.ds(zbase + 7 * _CH, rem)])
        plsc.subcore_barrier()

        def block(jj, carry0):
            pltpu.sync_copy(eidx_hbm.at[0, wid, jj], src_v)
            pltpu.sync_copy(eidx_hbm.at[1, wid, jj], dst_v)
            issue_gathers(0, 0)
            issue_gathers(1, 1)

            def pair(i, carry):
                a = 2 * i
                b = a + 1
                wait_gathers(a, 0)
                compute(0)
                issue_scatter(a, 0)
                wait_gathers(b, 1)
                compute(1)
                issue_scatter(b, 1)
                wait_scatter(a, 0)
                issue_gathers(a + 2, 0)
                wait_scatter(b, 1)
                issue_gathers(jnp.minimum(b + 2, _CPB - 1), 1)
                return carry

            lax.fori_loop(0, _NPAIR, pair, 0)
            last = _CPB - 1
            wait_gathers(last, 0)
            compute(0)
            pltpu.sync_copy(rows_v.at[0], agg_sh.at[dst_v.at[last]], add=True)
            wait_gathers(last, 1)
            return carry0

        lax.fori_loop(0, _NBLK, block, 0)
        plsc.subcore_barrier()
        pltpu.sync_copy(agg_sh.at[pl.ds(s * _RPT, _RPT)],
                        out_hbm.at[c, pl.ds(s * _RPT, _RPT)])

    return k(xm, a_src, a_dst, eidx)


def _tc_post_body(p_ref, base_ref, out_ref):
    out_ref[...] = jnp.maximum(p_ref[0] + p_ref[1] + base_ref[...], 0.0)


def _tc_post(partials, base):
    return pl.pallas_call(
        _tc_post_body,
        grid=(_GRID,),
        in_specs=[
            pl.BlockSpec((_NC, _BN, _D), lambda i: (0, i, 0)),
            pl.BlockSpec((_BN, _D), lambda i: (i, 0)),
        ],
        out_specs=pl.BlockSpec((_BN, _D), lambda i: (i, 0)),
        out_shape=jax.ShapeDtypeStruct((_N, _D), jnp.float32),
    )(partials, base)


def kernel(x, edge_index, Wa_w, Wa_b, Wm_w, Wm_b, Wr_w, Wr_b,
           bn_gamma, bn_beta, bn_mean, bn_var):
    wmt = Wm_w.T
    wrt = Wr_w.T
    wat = Wa_w.reshape(2, _D).T
    wab = jnp.concatenate([Wa_b, jnp.zeros((1,), jnp.float32)]).reshape(1, 2)
    xm, asd, base = _tc_pre(
        x, wmt, Wm_b.reshape(1, _D), wat, wab, wrt, Wr_b.reshape(1, _D),
        bn_gamma.reshape(1, _D), bn_beta.reshape(1, _D),
        bn_mean.reshape(1, _D), bn_var.reshape(1, _D))
    partials = _sc_edges(xm, asd[:, 0], asd[:, 1],
                         edge_index.reshape(2, _NW, _NBLK, _CPB, _CH))
    return _tc_post(partials, base)


# speedup vs baseline: 12.3264x; 1.0977x over previous
import functools

import jax
import jax.numpy as jnp
from jax import lax
from jax.experimental import pallas as pl
from jax.experimental.pallas import tpu as pltpu
from jax.experimental.pallas import tpu_sc as plsc

_N = 10000
_E = 320000
_D = 128
_EPS = 1e-5
_NC, _NS, _L = 2, 16, 16
_NW = _NC * _NS
_EPW = _E // _NW
_CH = 80
_NCHUNK = _EPW // _CH
_CPB = 25
_NBLK = _NCHUNK // _CPB
_NPAIR = (_CPB - 1) // 2
_RPT = 632
_NPAD = _NS * _RPT
_BN = 1000
_GRID = _N // _BN


def _tc_pre_body(x_ref, wmt_ref, wmb_ref, wat_ref, wab_ref, wrt_ref, wrb_ref,
                 gam_ref, bet_ref, mu_ref, var_ref,
                 xm_ref, asd_ref, base_ref):
    x = x_ref[...]
    k = gam_ref[...] * lax.rsqrt(var_ref[...] + _EPS)
    xm = jnp.dot(x, wmt_ref[...], preferred_element_type=jnp.float32)
    xm_ref[...] = (xm + wmb_ref[...]) * k
    asd_ref[...] = jnp.dot(x, wat_ref[...], preferred_element_type=jnp.float32) + wab_ref[...]
    xr = jnp.dot(x, wrt_ref[...], preferred_element_type=jnp.float32)
    base_ref[...] = (xr + wrb_ref[...] - mu_ref[...]) * k + bet_ref[...]


def _tc_pre(x, wmt, wmb, wat, wab, wrt, wrb, gam, bet, mu, var):
    full = lambda shape: pl.BlockSpec(shape, lambda i: (0, 0))
    blk = lambda shape: pl.BlockSpec(shape, lambda i: (i, 0))
    return pl.pallas_call(
        _tc_pre_body,
        grid=(_GRID,),
        in_specs=[
            blk((_BN, _D)),
            full((_D, _D)), full((1, _D)),
            full((_D, 2)), full((1, 2)),
            full((_D, _D)), full((1, _D)),
            full((1, _D)), full((1, _D)), full((1, _D)), full((1, _D)),
        ],
        out_specs=[blk((_BN, _D)), blk((_BN, 2)), blk((_BN, _D))],
        out_shape=[
            jax.ShapeDtypeStruct((_N, _D), jnp.float32),
            jax.ShapeDtypeStruct((_N, 2), jnp.float32),
            jax.ShapeDtypeStruct((_N, _D), jnp.float32),
        ],
    )(x, wmt, wmb, wat, wab, wrt, wrb, gam, bet, mu, var)


_GDN = lax.GatherDimensionNumbers(offset_dims=(), collapsed_slice_dims=(0,),
                                  start_index_map=(0,))


def _lane_bcast(vec, t):
    idx = jnp.full((_L, 1), t, jnp.int32)
    return lax.gather(vec, idx, _GDN, slice_sizes=(1,),
                      mode=lax.GatherScatterMode.PROMISE_IN_BOUNDS)


def _sc_edges(xm, a_src, a_dst, eidx):
    mesh = plsc.VectorSubcoreMesh(core_axis_name="c", subcore_axis_name="s",
                                  num_cores=_NC, num_subcores=_NS)

    @functools.partial(
        pl.kernel,
        out_type=jax.ShapeDtypeStruct((_NC, _NPAD, _D), jnp.float32),
        mesh=mesh,
        scratch_types=[
            pltpu.VMEM((_CPB, _CH), jnp.int32),
            pltpu.VMEM((_CPB, _CH), jnp.int32),
            pltpu.VMEM((2, _CH, _D), jnp.float32),
            pltpu.VMEM((2, _CH), jnp.float32),
            pltpu.VMEM((2, _CH), jnp.float32),
            pltpu.VMEM_SHARED((_NPAD, _D), jnp.float32),
            pltpu.SemaphoreType.DMA,
            pltpu.SemaphoreType.DMA,
            pltpu.SemaphoreType.DMA,
            pltpu.SemaphoreType.DMA,
        ],
        compiler_params=pltpu.CompilerParams(needs_layout_passes=False),
    )
    def k(xm_hbm, as_hbm, ad_hbm, eidx_hbm, out_hbm,
          src_v, dst_v, rows_v, ga_v, gb_v, agg_sh,
          gsem0, gsem1, ssem0, ssem1):
        c = lax.axis_index("c")
        s = lax.axis_index("s")
        wid = s * _NC + c
        gsems = (gsem0, gsem1)
        ssems = (ssem0, ssem1)

        def issue_gathers(j, p):
            pltpu.async_copy(xm_hbm.at[src_v.at[j]], rows_v.at[p], gsems[p])
            pltpu.async_copy(as_hbm.at[src_v.at[j]], ga_v.at[p], gsems[p])
            pltpu.async_copy(ad_hbm.at[dst_v.at[j]], gb_v.at[p], gsems[p])

        def wait_gathers(j, p):
            pltpu.make_async_copy(xm_hbm.at[src_v.at[j]], rows_v.at[p],
                                  gsems[p]).wait()
            pltpu.make_async_copy(as_hbm.at[src_v.at[j]], ga_v.at[p],
                                  gsems[p]).wait()
            pltpu.make_async_copy(ad_hbm.at[dst_v.at[j]], gb_v.at[p],
                                  gsems[p]).wait()

        def issue_scatter(j, p):
            pltpu.async_copy(rows_v.at[p], agg_sh.at[dst_v.at[j]], ssems[p],
                             add=True)

        def wait_scatter(j, p):
            pltpu.make_async_copy(rows_v.at[p], agg_sh.at[dst_v.at[j]],
                                  ssems[p]).wait()

        def compute(p):
            def group(v, carry):
                sl = pl.ds(v * _L, _L)
                g = 1.0 / (1.0 + jnp.exp(-(ga_v[p, sl] + gb_v[p, sl])))
                for t in range(_L):
                    gvec = _lane_bcast(g, t)
                    r = v * _L + t
                    for cc in range(_D // _L):
                        csl = pl.ds(cc * _L, _L)
                        rows_v[p, r, csl] = rows_v[p, r, csl] * gvec
                return carry

            lax.fori_loop(0, _CH // _L, group, 0)

        zero = jnp.zeros((_L,), jnp.float32)
        for r in range(_CH):
            for cc in range(_D // _L):
                rows_v[0, r, pl.ds(cc * _L, _L)] = zero
        zbase = s * _RPT
        for i in range(7):
            pltpu.sync_copy(rows_v.at[0], agg_sh.at[pl.ds(zbase + i * _CH, _CH)])
        rem = _RPT - 7 * _CH
        pltpu.sync_copy(rows_v.at[0].at[pl.ds(0, rem)],
                        agg_sh.at[pl.ds(zbase + 7 * _CH, rem)])
        plsc.subcore_barrier()

        def block(jj, carry0):
            pltpu.sync_copy(eidx_hbm.at[0, wid, jj], src_v)
            pltpu.sync_copy(eidx_hbm.at[1, wid, jj], dst_v)
            issue_gathers(0, 0)
            issue_gathers(1, 1)

            def pair(i, carry):
                a = 2 * i
                b = a + 1
                wait_gathers(a, 0)
                compute(0)
                issue_scatter(a, 0)
                wait_gathers(b, 1)
                compute(1)
                issue_scatter(b, 1)
                wait_scatter(a, 0)
                issue_gathers(a + 2, 0)
                wait_scatter(b, 1)
                issue_gathers(jnp.minimum(b + 2, _CPB - 1), 1)
                return carry

            lax.fori_loop(0, _NPAIR, pair, 0)
            last = _CPB - 1
            wait_gathers(last, 0)
            compute(0)
            pltpu.sync_copy(rows_v.at[0], agg_sh.at[dst_v.at[last]], add=True)
            wait_gathers(last, 1)
            return carry0

        lax.fori_loop(0, _NBLK, block, 0)
        plsc.subcore_barrier()
        pltpu.sync_copy(agg_sh.at[pl.ds(s * _RPT, _RPT)],
                        out_hbm.at[c, pl.ds(s * _RPT, _RPT)])

    return k(xm, a_src, a_dst, eidx)


def _tc_post_body(p_ref, base_ref, out_ref):
    out_ref[...] = jnp.maximum(p_ref[0] + p_ref[1] + base_ref[...], 0.0)


def _tc_post(partials, base):
    return pl.pallas_call(
        _tc_post_body,
        grid=(_GRID,),
        in_specs=[
            pl.BlockSpec((_NC, _BN, _D), lambda i: (0, i, 0)),
            pl.BlockSpec((_BN, _D), lambda i: (i, 0)),
        ],
        out_specs=pl.BlockSpec((_BN, _D), lambda i: (i, 0)),
        out_shape=jax.ShapeDtypeStruct((_N, _D), jnp.float32),
    )(partials, base)


def kernel(x, edge_index, Wa_w, Wa_b, Wm_w, Wm_b, Wr_w, Wr_b,
           bn_gamma, bn_beta, bn_mean, bn_var):
    wmt = Wm_w.T
    wrt = Wr_w.T
    wat = Wa_w.reshape(2, _D).T
    wab = jnp.concatenate([Wa_b, jnp.zeros((1,), jnp.float32)]).reshape(1, 2)
    xm, asd, base = _tc_pre(
        x, wmt, Wm_b.reshape(1, _D), wat, wab, wrt, Wr_b.reshape(1, _D),
        bn_gamma.reshape(1, _D), bn_beta.reshape(1, _D),
        bn_mean.reshape(1, _D), bn_var.reshape(1, _D))
    partials = _sc_edges(xm, asd[:, 0], asd[:, 1],
                         edge_index.reshape(2, _NW, _NBLK, _CPB, _CH))
    return _tc_post(partials, base)


# speedup vs baseline: 14.1696x; 1.1495x over previous
import functools

import jax
import jax.numpy as jnp
from jax import lax
from jax.experimental import pallas as pl
from jax.experimental.pallas import tpu as pltpu
from jax.experimental.pallas import tpu_sc as plsc

_N = 10000
_E = 320000
_D = 128
_EPS = 1e-5
_NC, _NS, _L = 2, 16, 16
_NW = _NC * _NS
_EPW = _E // _NW
_CH = 80
_NCHUNK = _EPW // _CH
_CPB = 25
_NBLK = _NCHUNK // _CPB
_NPAIR = (_CPB - 1) // 2
_RPT = 632
_NPAD = _NS * _RPT
_BN = 1000
_GRID = _N // _BN


def _tc_pre_body(x_ref, wmt_ref, wmb_ref, wat_ref, wab_ref, wrt_ref, wrb_ref,
                 gam_ref, bet_ref, mu_ref, var_ref,
                 xm_ref, asd_ref, base_ref):
    x = x_ref[...]
    k = gam_ref[...] * lax.rsqrt(var_ref[...] + _EPS)
    xm = jnp.dot(x, wmt_ref[...], preferred_element_type=jnp.float32)
    xm_ref[...] = (xm + wmb_ref[...]) * k
    asd_ref[...] = jnp.dot(x, wat_ref[...], preferred_element_type=jnp.float32) + wab_ref[...]
    xr = jnp.dot(x, wrt_ref[...], preferred_element_type=jnp.float32)
    base_ref[...] = (xr + wrb_ref[...] - mu_ref[...]) * k + bet_ref[...]


def _tc_pre(x, wmt, wmb, wat, wab, wrt, wrb, gam, bet, mu, var):
    full = lambda shape: pl.BlockSpec(shape, lambda i: (0, 0))
    blk = lambda shape: pl.BlockSpec(shape, lambda i: (i, 0))
    return pl.pallas_call(
        _tc_pre_body,
        grid=(_GRID,),
        in_specs=[
            blk((_BN, _D)),
            full((_D, _D)), full((1, _D)),
            full((_D, 2)), full((1, 2)),
            full((_D, _D)), full((1, _D)),
            full((1, _D)), full((1, _D)), full((1, _D)), full((1, _D)),
        ],
        out_specs=[blk((_BN, _D)), blk((_BN, 2)), blk((_BN, _D))],
        out_shape=[
            jax.ShapeDtypeStruct((_N, _D), jnp.float32),
            jax.ShapeDtypeStruct((_N, 2), jnp.float32),
            jax.ShapeDtypeStruct((_N, _D), jnp.float32),
        ],
    )(x, wmt, wmb, wat, wab, wrt, wrb, gam, bet, mu, var)


_GDN = lax.GatherDimensionNumbers(offset_dims=(), collapsed_slice_dims=(0,),
                                  start_index_map=(0,))


def _lane_bcast(vec, t):
    idx = jnp.full((_L, 1), t, jnp.int32)
    return lax.gather(vec, idx, _GDN, slice_sizes=(1,),
                      mode=lax.GatherScatterMode.PROMISE_IN_BOUNDS)


def _sc_edges(xm, a_src, a_dst, eidx):
    mesh = plsc.VectorSubcoreMesh(core_axis_name="c", subcore_axis_name="s",
                                  num_cores=_NC, num_subcores=_NS)

    @functools.partial(
        pl.kernel,
        out_type=jax.ShapeDtypeStruct((_NC, _NPAD, _D), jnp.float32),
        mesh=mesh,
        scratch_types=[
            pltpu.VMEM((_CPB, _CH), jnp.int32),
            pltpu.VMEM((_CPB, _CH), jnp.int32),
            pltpu.VMEM((3, _CH, _D), jnp.float32),
            pltpu.VMEM((3, _CH), jnp.float32),
            pltpu.VMEM((3, _CH), jnp.float32),
            pltpu.VMEM_SHARED((_NPAD, _D), jnp.float32),
            pltpu.SemaphoreType.DMA,
            pltpu.SemaphoreType.DMA,
            pltpu.SemaphoreType.DMA,
            pltpu.SemaphoreType.DMA,
            pltpu.SemaphoreType.DMA,
            pltpu.SemaphoreType.DMA,
        ],
        compiler_params=pltpu.CompilerParams(needs_layout_passes=False),
    )
    def k(xm_hbm, as_hbm, ad_hbm, eidx_hbm, out_hbm,
          src_v, dst_v, rows_v, ga_v, gb_v, agg_sh,
          gsem0, gsem1, gsem2, ssem0, ssem1, ssem2):
        c = lax.axis_index("c")
        s = lax.axis_index("s")
        wid = s * _NC + c
        gsems = (gsem0, gsem1, gsem2)
        ssems = (ssem0, ssem1, ssem2)

        def issue_gathers(j, p):
            pltpu.async_copy(xm_hbm.at[src_v.at[j]], rows_v.at[p], gsems[p])
            pltpu.async_copy(as_hbm.at[src_v.at[j]], ga_v.at[p], gsems[p])
            pltpu.async_copy(ad_hbm.at[dst_v.at[j]], gb_v.at[p], gsems[p])

        def wait_gathers(j, p):
            pltpu.make_async_copy(xm_hbm.at[src_v.at[j]], rows_v.at[p],
                                  gsems[p]).wait()
            pltpu.make_async_copy(as_hbm.at[src_v.at[j]], ga_v.at[p],
                                  gsems[p]).wait()
            pltpu.make_async_copy(ad_hbm.at[dst_v.at[j]], gb_v.at[p],
                                  gsems[p]).wait()

        def issue_scatter(j, p):
            pltpu.async_copy(rows_v.at[p], agg_sh.at[dst_v.at[j]], ssems[p],
                             add=True)

        def wait_scatter(j, p):
            pltpu.make_async_copy(rows_v.at[p], agg_sh.at[dst_v.at[j]],
                                  ssems[p]).wait()

        def compute(p):
            def group(v, carry):
                sl = pl.ds(v * _L, _L)
                g = 1.0 / (1.0 + jnp.exp(-(ga_v[p, sl] + gb_v[p, sl])))
                for t in range(_L):
                    gvec = _lane_bcast(g, t)
                    r = v * _L + t
                    for cc in range(_D // _L):
                        csl = pl.ds(cc * _L, _L)
                        rows_v[p, r, csl] = rows_v[p, r, csl] * gvec
                return carry

            lax.fori_loop(0, _CH // _L, group, 0)

        zero = jnp.zeros((_L,), jnp.float32)
        for r in range(_CH):
            for cc in range(_D // _L):
                rows_v[0, r, pl.ds(cc * _L, _L)] = zero
        zbase = s * _RPT
        for i in range(7):
            pltpu.sync_copy(rows_v.at[0], agg_sh.at[pl.ds(zbase + i * _CH, _CH)])
        rem = _RPT - 7 * _CH
        pltpu.sync_copy(rows_v.at[0].at[pl.ds(0, rem)],
                        agg_sh.at[pl.ds(zbase + 7 * _CH, rem)])
        plsc.subcore_barrier()

        def step(j, p, pn):
            wait_gathers(j, p)
            compute(p)
            issue_scatter(j, p)
            wait_scatter(j - 1, pn)
            issue_gathers(j + 2, pn)

        def block(jj, carry0):
            pltpu.sync_copy(eidx_hbm.at[0, wid, jj], src_v)
            pltpu.sync_copy(eidx_hbm.at[1, wid, jj], dst_v)
            issue_gathers(0, 0)
            issue_gathers(1, 1)
            wait_gathers(0, 0)
            compute(0)
            issue_scatter(0, 0)
            issue_gathers(2, 2)
            step(1, 1, 0)

            def triple(t, carry):
                j0 = 3 * t + 2
                step(j0, 2, 1)
                step(j0 + 1, 0, 2)
                step(j0 + 2, 1, 0)
                return carry

            lax.fori_loop(0, 7, triple, 0)
            wait_gathers(23, 2)
            compute(2)
            issue_scatter(23, 2)
            wait_scatter(22, 1)
            wait_gathers(24, 0)
            compute(0)
            issue_scatter(24, 0)
            wait_scatter(23, 2)
            wait_scatter(24, 0)
            return carry0

        lax.fori_loop(0, _NBLK, block, 0)
        plsc.subcore_barrier()
        pltpu.sync_copy(agg_sh.at[pl.ds(s * _RPT, _RPT)],
                        out_hbm.at[c, pl.ds(s * _RPT, _RPT)])

    return k(xm, a_src, a_dst, eidx)


def _tc_post_body(p_ref, base_ref, out_ref):
    out_ref[...] = jnp.maximum(p_ref[0] + p_ref[1] + base_ref[...], 0.0)


def _tc_post(partials, base):
    return pl.pallas_call(
        _tc_post_body,
        grid=(_GRID,),
        in_specs=[
            pl.BlockSpec((_NC, _BN, _D), lambda i: (0, i, 0)),
            pl.BlockSpec((_BN, _D), lambda i: (i, 0)),
        ],
        out_specs=pl.BlockSpec((_BN, _D), lambda i: (i, 0)),
        out_shape=jax.ShapeDtypeStruct((_N, _D), jnp.float32),
    )(partials, base)


def kernel(x, edge_index, Wa_w, Wa_b, Wm_w, Wm_b, Wr_w, Wr_b,
           bn_gamma, bn_beta, bn_mean, bn_var):
    wmt = Wm_w.T
    wrt = Wr_w.T
    wat = Wa_w.reshape(2, _D).T
    wab = jnp.concatenate([Wa_b, jnp.zeros((1,), jnp.float32)]).reshape(1, 2)
    xm, asd, base = _tc_pre(
        x, wmt, Wm_b.reshape(1, _D), wat, wab, wrt, Wr_b.reshape(1, _D),
        bn_gamma.reshape(1, _D), bn_beta.reshape(1, _D),
        bn_mean.reshape(1, _D), bn_var.reshape(1, _D))
    partials = _sc_edges(xm, asd[:, 0], asd[:, 1],
                         edge_index.reshape(2, _NW, _NBLK, _CPB, _CH))
    return _tc_post(partials, base)
